# Initial kernel scaffold; baseline (speedup 1.0000x reference)
import jax
import jax.numpy as jnp
from jax.experimental import pallas as pl


def kernel(x, edge_index, W, b):
    raise NotImplementedError("write your pallas kernel here")


# speedup vs baseline: 21.6033x; 21.6033x over previous
import functools

import jax
import jax.numpy as jnp
from jax import lax
from jax.experimental import pallas as pl
from jax.experimental.pallas import tpu as pltpu
from jax.experimental.pallas import tpu_sc as plsc

N_NODES = 50000
N_EDGES = 800000
D_IN = 64
D_OUT = 64
H = 32
NC, NS = 2, 16
LANE = 128
EROWS = 6400
EPAD = EROWS * LANE - N_EDGES
NPAD = 50016
SCRAP = NPAD - 1
WU = 391
SROWS = WU * 8
DEG_W = 8
TROWS = EROWS // NS
BLK = 16
NBLK = TROWS // BLK
DROWS = EROWS // (NC * NS)
DBLK = 8
DNBLK = DROWS // DBLK
NRING = 4

_mesh = plsc.VectorSubcoreMesh(
    core_axis_name="c", subcore_axis_name="s", num_cores=NC, num_subcores=NS)
_sc_params = pltpu.CompilerParams(use_tc_tiling_on_sc=False)


def _tile_base_rows(s):
    return 8 * (s * 390 + lax.min(s, 11))


@functools.partial(
    pl.kernel,
    out_type=jax.ShapeDtypeStruct((NC * NPAD, DEG_W), jnp.float32),
    mesh=_mesh,
    scratch_types=[
        pltpu.VMEM((DBLK, LANE), jnp.int32),
        pltpu.VMEM((LANE, DEG_W), jnp.float32),
        pltpu.SemaphoreType.DMA,
        pltpu.VMEM_SHARED((NPAD, DEG_W), jnp.float32),
    ],
    compiler_params=_sc_params,
)
def _deg_kernel(dst_hbm, ones_hbm, zeros_hbm, deg_hbm,
                dstb, ones_v, sem, deg_s):
    c = lax.axis_index("c")
    s = lax.axis_index("s")
    wid = c * NS + s
    base_rows = _tile_base_rows(s)
    pltpu.sync_copy(ones_hbm, ones_v)
    pltpu.sync_copy(zeros_hbm, deg_s.at[pl.ds(base_rows, SROWS)])
    plsc.subcore_barrier()
    wbase = wid * DROWS

    def body(k, carry):
        pltpu.sync_copy(dst_hbm.at[pl.ds(wbase + k * DBLK, DBLK)], dstb)
        descs = [
            pltpu.async_copy(ones_v, deg_s.at[dstb.at[j]], sem, add=True)
            for j in range(DBLK)
        ]
        for d in descs:
            d.wait()
        return carry

    lax.fori_loop(0, DNBLK, body, 0)
    plsc.subcore_barrier()
    pltpu.sync_copy(deg_s.at[pl.ds(base_rows, SROWS)],
                    deg_hbm.at[pl.ds(c * NPAD + base_rows, SROWS)])


@functools.partial(
    pl.kernel,
    out_type=jax.ShapeDtypeStruct((NC * NPAD, H), jnp.float32),
    mesh=_mesh,
    scratch_types=[
        pltpu.VMEM((BLK, LANE), jnp.int32),
        pltpu.VMEM((BLK, LANE), jnp.int32),
        [pltpu.VMEM((LANE, H), jnp.float32)] * NRING,
        [pltpu.SemaphoreType.DMA] * NRING,
        [pltpu.SemaphoreType.DMA] * NRING,
        pltpu.VMEM_SHARED((NPAD, H), jnp.float32),
    ],
    compiler_params=_sc_params,
)
def _edge_kernel(src_hbm, dst_hbm, h2_hbm, zeros_hbm, acc_hbm,
                 srcb, dstb, rows, gsems, ssems, acc_s):
    c = lax.axis_index("c")
    s = lax.axis_index("s")
    base_rows = _tile_base_rows(s)
    pltpu.sync_copy(zeros_hbm, acc_s.at[pl.ds(base_rows, SROWS)])
    plsc.subcore_barrier()
    tb = s * TROWS

    def body(k, carry):
        blk = tb + k * BLK
        pltpu.sync_copy(src_hbm.at[pl.ds(c * EROWS + blk, BLK)], srcb)
        pltpu.sync_copy(dst_hbm.at[pl.ds(blk, BLK)], dstb)
        gd = [None] * NRING
        sd = [None] * NRING
        for t in range(BLK + NRING - 1):
            slot = t % NRING
            if t < BLK:
                if sd[slot] is not None:
                    sd[slot].wait()
                    sd[slot] = None
                gd[slot] = pltpu.async_copy(
                    h2_hbm.at[srcb.at[t]], rows[slot], gsems[slot])
            if t >= NRING - 1:
                r = t - (NRING - 1)
                rs = r % NRING
                gd[rs].wait()
                sd[rs] = pltpu.async_copy(
                    rows[rs], acc_s.at[dstb.at[r]], ssems[rs], add=True)
        for slot in range(NRING):
            if sd[slot] is not None:
                sd[slot].wait()
        return carry

    lax.fori_loop(0, NBLK, body, 0)
    plsc.subcore_barrier()
    pltpu.sync_copy(acc_s.at[pl.ds(base_rows, SROWS)],
                    acc_hbm.at[pl.ds(c * NPAD + base_rows, SROWS)])


_RB = 400
_NBLK_TC = N_NODES // _RB


def _mm_body(x_ref, w_ref, d0_ref, d1_ref, h2_ref, dis_ref):
    deg = d0_ref[:, 0:1] + d1_ref[:, 0:1] + 1.0
    dis = lax.rsqrt(deg)
    h = jnp.dot(x_ref[...], w_ref[...], preferred_element_type=jnp.float32)
    h2 = h * dis
    h2_ref[0, :, :] = h2[:, :H]
    h2_ref[1, :, :] = h2[:, H:]
    dis_ref[...] = dis


def _combine_body(a0_ref, a1_ref, h2_ref, dis_ref, b_ref, out_ref):
    dis = dis_ref[...]
    o0 = dis * (a0_ref[...] + h2_ref[0]) + b_ref[0]
    o1 = dis * (a1_ref[...] + h2_ref[1]) + b_ref[1]
    out_ref[...] = jnp.concatenate([o0, o1], axis=1)


def kernel(x, edge_index, W, b):
    src_pad = jnp.concatenate(
        [edge_index[0], jnp.zeros((EPAD,), jnp.int32)])
    dst_pad = jnp.concatenate(
        [edge_index[1], jnp.full((EPAD,), SCRAP, jnp.int32)])
    src01 = jnp.concatenate(
        [src_pad, src_pad + N_NODES]).reshape(NC * EROWS, LANE)
    dst2 = dst_pad.reshape(EROWS, LANE)
    ones_deg = jnp.ones((LANE, DEG_W), jnp.float32)
    zeros_deg = jnp.zeros((SROWS, DEG_W), jnp.float32)
    zeros_acc = jnp.zeros((SROWS, H), jnp.float32)

    degflat = _deg_kernel(dst2, ones_deg, zeros_deg)
    d0 = lax.slice(degflat, (0, 0), (N_NODES, DEG_W))
    d1 = lax.slice(degflat, (NPAD, 0), (NPAD + N_NODES, DEG_W))

    h2pair, dis = pl.pallas_call(
        _mm_body,
        grid=(_NBLK_TC,),
        in_specs=[
            pl.BlockSpec((_RB, D_IN), lambda i: (i, 0)),
            pl.BlockSpec((D_IN, D_OUT), lambda i: (0, 0)),
            pl.BlockSpec((_RB, DEG_W), lambda i: (i, 0)),
            pl.BlockSpec((_RB, DEG_W), lambda i: (i, 0)),
        ],
        out_specs=[
            pl.BlockSpec((2, _RB, H), lambda i: (0, i, 0)),
            pl.BlockSpec((_RB, 1), lambda i: (i, 0)),
        ],
        out_shape=[
            jax.ShapeDtypeStruct((2, N_NODES, H), jnp.float32),
            jax.ShapeDtypeStruct((N_NODES, 1), jnp.float32),
        ],
    )(x, W, d0, d1)

    h2flat = h2pair.reshape(2 * N_NODES, H)
    accflat = _edge_kernel(src01, dst2, h2flat, zeros_acc)
    a0 = lax.slice(accflat, (0, 0), (N_NODES, H))
    a1 = lax.slice(accflat, (NPAD, 0), (NPAD + N_NODES, H))
    bpair = b.reshape(2, H)[:, None, :]

    out = pl.pallas_call(
        _combine_body,
        grid=(_NBLK_TC,),
        in_specs=[
            pl.BlockSpec((_RB, H), lambda i: (i, 0)),
            pl.BlockSpec((_RB, H), lambda i: (i, 0)),
            pl.BlockSpec((2, _RB, H), lambda i: (0, i, 0)),
            pl.BlockSpec((_RB, 1), lambda i: (i, 0)),
            pl.BlockSpec((2, 1, H), lambda i: (0, 0, 0)),
        ],
        out_specs=pl.BlockSpec((_RB, D_OUT), lambda i: (i, 0)),
        out_shape=jax.ShapeDtypeStruct((N_NODES, D_OUT), jnp.float32),
    )(a0, a1, h2pair, dis, bpair)
    return out


# speedup vs baseline: 26.2615x; 1.2156x over previous
import functools

import jax
import jax.numpy as jnp
from jax import lax
from jax.experimental import pallas as pl
from jax.experimental.pallas import tpu as pltpu
from jax.experimental.pallas import tpu_sc as plsc

N_NODES = 50000
N_EDGES = 800000
D_IN = 64
D_OUT = 64
H = 32
NC, NS = 2, 16
LANE = 128
EROWS = 6400
EPAD = EROWS * LANE - N_EDGES
NPAD = 50048
SCRAP = NPAD - 1
SROWS = NPAD // NS
DEG_W = 8
TROWS = EROWS // NS
BLK = 8
NBODY = TROWS // (2 * BLK)
DROWS = EROWS // (NC * NS)
DNBLK = DROWS // BLK
NRING = 4

_mesh = plsc.VectorSubcoreMesh(
    core_axis_name="c", subcore_axis_name="s", num_cores=NC, num_subcores=NS)
_sc_params = pltpu.CompilerParams(use_tc_tiling_on_sc=False)


@functools.partial(
    pl.kernel,
    out_type=(jax.ShapeDtypeStruct((NPAD, DEG_W), jnp.float32),
              jax.ShapeDtypeStruct((NPAD, DEG_W), jnp.float32)),
    mesh=_mesh,
    scratch_types=[
        pltpu.VMEM((BLK, LANE), jnp.int32),
        pltpu.VMEM((LANE, DEG_W), jnp.float32),
        pltpu.SemaphoreType.DMA,
        pltpu.VMEM_SHARED((NPAD, DEG_W), jnp.float32),
    ],
    compiler_params=_sc_params,
)
def _deg_kernel(dst_hbm, ones_hbm, zeros_hbm, deg0_hbm, deg1_hbm,
                dstb, ones_v, sem, deg_s):
    c = lax.axis_index("c")
    s = lax.axis_index("s")
    wid = c * NS + s
    base_rows = s * SROWS
    pltpu.sync_copy(ones_hbm, ones_v)
    pltpu.sync_copy(zeros_hbm, deg_s.at[pl.ds(base_rows, SROWS)])
    plsc.subcore_barrier()
    wbase = wid * DROWS

    def body(k, carry):
        pltpu.sync_copy(dst_hbm.at[pl.ds(wbase + k * BLK, BLK)], dstb)
        descs = [
            pltpu.async_copy(ones_v, deg_s.at[dstb.at[j]], sem, add=True)
            for j in range(BLK)
        ]
        for d in descs:
            d.wait()
        return carry

    lax.fori_loop(0, DNBLK, body, 0)
    plsc.subcore_barrier()

    @pl.when(c == 0)
    def _():
        pltpu.sync_copy(deg_s.at[pl.ds(base_rows, SROWS)],
                        deg0_hbm.at[pl.ds(base_rows, SROWS)])

    @pl.when(c == 1)
    def _():
        pltpu.sync_copy(deg_s.at[pl.ds(base_rows, SROWS)],
                        deg1_hbm.at[pl.ds(base_rows, SROWS)])


@functools.partial(
    pl.kernel,
    out_type=(jax.ShapeDtypeStruct((NPAD, H), jnp.float32),
              jax.ShapeDtypeStruct((NPAD, H), jnp.float32)),
    mesh=_mesh,
    scratch_types=[
        [pltpu.VMEM((BLK, LANE), jnp.int32)] * 2,
        [pltpu.VMEM((BLK, LANE), jnp.int32)] * 2,
        [pltpu.VMEM((LANE, H), jnp.float32)] * NRING,
        [pltpu.SemaphoreType.DMA] * 2,
        [pltpu.SemaphoreType.DMA] * NRING,
        [pltpu.SemaphoreType.DMA] * NRING,
        pltpu.VMEM_SHARED((NPAD, H), jnp.float32),
    ],
    compiler_params=_sc_params,
)
def _edge_kernel(src_hbm, dst_hbm, h2_hbm, zeros_hbm, acc0_hbm, acc1_hbm,
                 srcb, dstb, rows, isems, gsems, ssems, acc_s):
    c = lax.axis_index("c")
    s = lax.axis_index("s")
    base_rows = s * SROWS
    pltpu.sync_copy(zeros_hbm, acc_s.at[pl.ds(base_rows, SROWS)])
    plsc.subcore_barrier()
    tb = s * TROWS

    def iload(blk_idx, buf):
        base = blk_idx * BLK
        a = pltpu.async_copy(
            src_hbm.at[pl.ds(c * EROWS + tb + base, BLK)], srcb[buf],
            isems[buf])
        b = pltpu.async_copy(
            dst_hbm.at[pl.ds(tb + base, BLK)], dstb[buf], isems[buf])
        return a, b

    def iwait(buf):
        pltpu.make_async_copy(
            src_hbm.at[pl.ds(tb, BLK)], srcb[buf], isems[buf]).wait()
        pltpu.make_async_copy(
            dst_hbm.at[pl.ds(tb, BLK)], dstb[buf], isems[buf]).wait()

    iload(0, 0)
    iload(1, 1)

    def body(m, carry):
        iwait(0)
        gd = [None] * NRING
        sd = [None] * NRING
        nrows = 2 * BLK
        for t in range(nrows + NRING - 1):
            slot = t % NRING
            if t == BLK:
                iwait(1)
            if t < nrows:
                if sd[slot] is not None:
                    sd[slot].wait()
                    sd[slot] = None
                srow = srcb[0].at[t] if t < BLK else srcb[1].at[t - BLK]
                gd[slot] = pltpu.async_copy(
                    h2_hbm.at[srow], rows[slot], gsems[slot])
            if t >= NRING - 1:
                r = t - (NRING - 1)
                rs = r % NRING
                gd[rs].wait()
                drow = dstb[0].at[r] if r < BLK else dstb[1].at[r - BLK]
                sd[rs] = pltpu.async_copy(
                    rows[rs], acc_s.at[drow], ssems[rs], add=True)
            if t == BLK + NRING - 1:
                @pl.when(m < NBODY - 1)
                def _():
                    iload(2 * m + 2, 0)
        for slot in range(NRING):
            if sd[slot] is not None:
                sd[slot].wait()

        @pl.when(m < NBODY - 1)
        def _():
            iload(2 * m + 3, 1)
        return carry

    lax.fori_loop(0, NBODY, body, 0)
    plsc.subcore_barrier()

    @pl.when(c == 0)
    def _():
        pltpu.sync_copy(acc_s.at[pl.ds(base_rows, SROWS)],
                        acc0_hbm.at[pl.ds(base_rows, SROWS)])

    @pl.when(c == 1)
    def _():
        pltpu.sync_copy(acc_s.at[pl.ds(base_rows, SROWS)],
                        acc1_hbm.at[pl.ds(base_rows, SROWS)])


_RB = 2000
_NBLK_TC = N_NODES // _RB


def _mm_body(x_ref, w_ref, d0_ref, d1_ref, h2_ref, dis_ref):
    deg = d0_ref[:, 0:1] + d1_ref[:, 0:1] + 1.0
    dis = lax.rsqrt(deg)
    h = jnp.dot(x_ref[...], w_ref[0], preferred_element_type=jnp.float32)
    h2_ref[...] = h * dis
    dis_ref[...] = dis


def _combine_body(a0_ref, a1_ref, h2a_ref, h2b_ref, dis_ref,
                  b0_ref, b1_ref, out_ref):
    dis = dis_ref[...]
    o0 = dis * (a0_ref[...] + h2a_ref[...]) + b0_ref[...]
    o1 = dis * (a1_ref[...] + h2b_ref[...]) + b1_ref[...]
    out_ref[...] = jnp.concatenate([o0, o1], axis=1)


def kernel(x, edge_index, W, b):
    src_pad = jnp.concatenate(
        [edge_index[0], jnp.zeros((EPAD,), jnp.int32)])
    dst_pad = jnp.concatenate(
        [edge_index[1], jnp.full((EPAD,), SCRAP, jnp.int32)])
    src01 = jnp.concatenate(
        [src_pad, src_pad + N_NODES]).reshape(NC * EROWS, LANE)
    dst2 = dst_pad.reshape(EROWS, LANE)
    wsplit = jnp.stack([W[:, :H], W[:, H:]])
    ones_deg = jnp.ones((LANE, DEG_W), jnp.float32)
    zeros_deg = jnp.zeros((SROWS, DEG_W), jnp.float32)
    zeros_acc = jnp.zeros((SROWS, H), jnp.float32)

    deg0, deg1 = _deg_kernel(dst2, ones_deg, zeros_deg)

    h2flat, dis = pl.pallas_call(
        _mm_body,
        grid=(2, _NBLK_TC),
        in_specs=[
            pl.BlockSpec((_RB, D_IN), lambda p, i: (i, 0)),
            pl.BlockSpec((1, D_IN, H), lambda p, i: (p, 0, 0)),
            pl.BlockSpec((_RB, DEG_W), lambda p, i: (i, 0)),
            pl.BlockSpec((_RB, DEG_W), lambda p, i: (i, 0)),
        ],
        out_specs=[
            pl.BlockSpec((_RB, H), lambda p, i: (p * _NBLK_TC + i, 0)),
            pl.BlockSpec((_RB, 1), lambda p, i: (i, 0)),
        ],
        out_shape=[
            jax.ShapeDtypeStruct((2 * N_NODES, H), jnp.float32),
            jax.ShapeDtypeStruct((N_NODES, 1), jnp.float32),
        ],
    )(x, wsplit, deg0, deg1)

    acc0, acc1 = _edge_kernel(src01, dst2, h2flat, zeros_acc)

    out = pl.pallas_call(
        _combine_body,
        grid=(_NBLK_TC,),
        in_specs=[
            pl.BlockSpec((_RB, H), lambda i: (i, 0)),
            pl.BlockSpec((_RB, H), lambda i: (i, 0)),
            pl.BlockSpec((_RB, H), lambda i: (i, 0)),
            pl.BlockSpec((_RB, H), lambda i: (i + _NBLK_TC, 0)),
            pl.BlockSpec((_RB, 1), lambda i: (i, 0)),
            pl.BlockSpec((1, H), lambda i: (0, 0)),
            pl.BlockSpec((1, H), lambda i: (0, 0)),
        ],
        out_specs=pl.BlockSpec((_RB, D_OUT), lambda i: (i, 0)),
        out_shape=jax.ShapeDtypeStruct((N_NODES, D_OUT), jnp.float32),
    )(acc0, acc1, h2flat, h2flat, dis,
      b[:H].reshape(1, H), b[H:].reshape(1, H))
    return out


# speedup vs baseline: 26.3631x; 1.0039x over previous
import functools

import jax
import jax.numpy as jnp
from jax import lax
from jax.experimental import pallas as pl
from jax.experimental.pallas import tpu as pltpu
from jax.experimental.pallas import tpu_sc as plsc

N_NODES = 50000
N_EDGES = 800000
D_IN = 64
D_OUT = 64
H = 32
NC, NS = 2, 16
LANE = 128
EROWS = 6400
EPAD = EROWS * LANE - N_EDGES
NPAD = 50048
SCRAP = NPAD - 1
SROWS = NPAD // NS
DEG_W = 8
TROWS = EROWS // NS
BLK = 8
NBODY = TROWS // (2 * BLK)
DROWS = EROWS // (NC * NS)
DNBLK = DROWS // BLK
NRING = 6

_mesh = plsc.VectorSubcoreMesh(
    core_axis_name="c", subcore_axis_name="s", num_cores=NC, num_subcores=NS)
_sc_params = pltpu.CompilerParams(use_tc_tiling_on_sc=False)


@functools.partial(
    pl.kernel,
    out_type=(jax.ShapeDtypeStruct((NPAD, DEG_W), jnp.float32),
              jax.ShapeDtypeStruct((NPAD, DEG_W), jnp.float32)),
    mesh=_mesh,
    scratch_types=[
        pltpu.VMEM((BLK, LANE), jnp.int32),
        pltpu.VMEM((LANE, DEG_W), jnp.float32),
        pltpu.SemaphoreType.DMA,
        pltpu.VMEM_SHARED((NPAD, DEG_W), jnp.float32),
    ],
    compiler_params=_sc_params,
)
def _deg_kernel(dst_hbm, ones_hbm, zeros_hbm, deg0_hbm, deg1_hbm,
                dstb, ones_v, sem, deg_s):
    c = lax.axis_index("c")
    s = lax.axis_index("s")
    wid = c * NS + s
    base_rows = s * SROWS
    pltpu.sync_copy(ones_hbm, ones_v)
    pltpu.sync_copy(zeros_hbm, deg_s.at[pl.ds(base_rows, SROWS)])
    plsc.subcore_barrier()
    wbase = wid * DROWS

    def body(k, carry):
        pltpu.sync_copy(dst_hbm.at[pl.ds(wbase + k * BLK, BLK)], dstb)
        descs = [
            pltpu.async_copy(ones_v, deg_s.at[dstb.at[j]], sem, add=True)
            for j in range(BLK)
        ]
        for d in descs:
            d.wait()
        return carry

    lax.fori_loop(0, DNBLK, body, 0)
    plsc.subcore_barrier()

    @pl.when(c == 0)
    def _():
        pltpu.sync_copy(deg_s.at[pl.ds(base_rows, SROWS)],
                        deg0_hbm.at[pl.ds(base_rows, SROWS)])

    @pl.when(c == 1)
    def _():
        pltpu.sync_copy(deg_s.at[pl.ds(base_rows, SROWS)],
                        deg1_hbm.at[pl.ds(base_rows, SROWS)])


@functools.partial(
    pl.kernel,
    out_type=(jax.ShapeDtypeStruct((NPAD, H), jnp.float32),
              jax.ShapeDtypeStruct((NPAD, H), jnp.float32)),
    mesh=_mesh,
    scratch_types=[
        [pltpu.VMEM((BLK, LANE), jnp.int32)] * 2,
        [pltpu.VMEM((BLK, LANE), jnp.int32)] * 2,
        [pltpu.VMEM((LANE, H), jnp.float32)] * NRING,
        [pltpu.SemaphoreType.DMA] * 2,
        [pltpu.SemaphoreType.DMA] * NRING,
        [pltpu.SemaphoreType.DMA] * NRING,
        pltpu.VMEM_SHARED((NPAD, H), jnp.float32),
    ],
    compiler_params=_sc_params,
)
def _edge_kernel(src_hbm, dst_hbm, h2_hbm, zeros_hbm, acc0_hbm, acc1_hbm,
                 srcb, dstb, rows, isems, gsems, ssems, acc_s):
    c = lax.axis_index("c")
    s = lax.axis_index("s")
    base_rows = s * SROWS
    pltpu.sync_copy(zeros_hbm, acc_s.at[pl.ds(base_rows, SROWS)])
    plsc.subcore_barrier()
    tb = s * TROWS

    def iload(blk_idx, buf):
        base = blk_idx * BLK
        a = pltpu.async_copy(
            src_hbm.at[pl.ds(c * EROWS + tb + base, BLK)], srcb[buf],
            isems[buf])
        b = pltpu.async_copy(
            dst_hbm.at[pl.ds(tb + base, BLK)], dstb[buf], isems[buf])
        return a, b

    def iwait(buf):
        pltpu.make_async_copy(
            src_hbm.at[pl.ds(tb, BLK)], srcb[buf], isems[buf]).wait()
        pltpu.make_async_copy(
            dst_hbm.at[pl.ds(tb, BLK)], dstb[buf], isems[buf]).wait()

    iload(0, 0)
    iload(1, 1)

    def body(m, carry):
        iwait(0)
        gd = [None] * NRING
        sd = [None] * NRING
        nrows = 2 * BLK
        for t in range(nrows + NRING - 1):
            slot = t % NRING
            if t == BLK:
                iwait(1)
            if t < nrows:
                if sd[slot] is not None:
                    sd[slot].wait()
                    sd[slot] = None
                srow = srcb[0].at[t] if t < BLK else srcb[1].at[t - BLK]
                gd[slot] = pltpu.async_copy(
                    h2_hbm.at[srow], rows[slot], gsems[slot])
            if t >= NRING - 1:
                r = t - (NRING - 1)
                rs = r % NRING
                gd[rs].wait()
                drow = dstb[0].at[r] if r < BLK else dstb[1].at[r - BLK]
                sd[rs] = pltpu.async_copy(
                    rows[rs], acc_s.at[drow], ssems[rs], add=True)
            if t == BLK + NRING - 1:
                @pl.when(m < NBODY - 1)
                def _():
                    iload(2 * m + 2, 0)
        for slot in range(NRING):
            if sd[slot] is not None:
                sd[slot].wait()

        @pl.when(m < NBODY - 1)
        def _():
            iload(2 * m + 3, 1)
        return carry

    lax.fori_loop(0, NBODY, body, 0)
    plsc.subcore_barrier()

    @pl.when(c == 0)
    def _():
        pltpu.sync_copy(acc_s.at[pl.ds(base_rows, SROWS)],
                        acc0_hbm.at[pl.ds(base_rows, SROWS)])

    @pl.when(c == 1)
    def _():
        pltpu.sync_copy(acc_s.at[pl.ds(base_rows, SROWS)],
                        acc1_hbm.at[pl.ds(base_rows, SROWS)])


_RB = 2000
_NBLK_TC = N_NODES // _RB


def _mm_body(x_ref, w_ref, d0_ref, d1_ref, h2_ref, dis_ref):
    deg = d0_ref[:, 0:1] + d1_ref[:, 0:1] + 1.0
    dis = lax.rsqrt(deg)
    h = jnp.dot(x_ref[...], w_ref[0], preferred_element_type=jnp.float32)
    h2_ref[...] = h * dis
    dis_ref[...] = dis


def _combine_body(a0_ref, a1_ref, h2a_ref, h2b_ref, dis_ref,
                  b0_ref, b1_ref, out_ref):
    dis = dis_ref[...]
    o0 = dis * (a0_ref[...] + h2a_ref[...]) + b0_ref[...]
    o1 = dis * (a1_ref[...] + h2b_ref[...]) + b1_ref[...]
    out_ref[...] = jnp.concatenate([o0, o1], axis=1)


def kernel(x, edge_index, W, b):
    src_pad = jnp.concatenate(
        [edge_index[0], jnp.zeros((EPAD,), jnp.int32)])
    dst_pad = jnp.concatenate(
        [edge_index[1], jnp.full((EPAD,), SCRAP, jnp.int32)])
    src01 = jnp.concatenate(
        [src_pad, src_pad + N_NODES]).reshape(NC * EROWS, LANE)
    dst2 = dst_pad.reshape(EROWS, LANE)
    wsplit = jnp.stack([W[:, :H], W[:, H:]])
    ones_deg = jnp.ones((LANE, DEG_W), jnp.float32)
    zeros_deg = jnp.zeros((SROWS, DEG_W), jnp.float32)
    zeros_acc = jnp.zeros((SROWS, H), jnp.float32)

    deg0, deg1 = _deg_kernel(dst2, ones_deg, zeros_deg)

    h2flat, dis = pl.pallas_call(
        _mm_body,
        grid=(2, _NBLK_TC),
        in_specs=[
            pl.BlockSpec((_RB, D_IN), lambda p, i: (i, 0)),
            pl.BlockSpec((1, D_IN, H), lambda p, i: (p, 0, 0)),
            pl.BlockSpec((_RB, DEG_W), lambda p, i: (i, 0)),
            pl.BlockSpec((_RB, DEG_W), lambda p, i: (i, 0)),
        ],
        out_specs=[
            pl.BlockSpec((_RB, H), lambda p, i: (p * _NBLK_TC + i, 0)),
            pl.BlockSpec((_RB, 1), lambda p, i: (i, 0)),
        ],
        out_shape=[
            jax.ShapeDtypeStruct((2 * N_NODES, H), jnp.float32),
            jax.ShapeDtypeStruct((N_NODES, 1), jnp.float32),
        ],
    )(x, wsplit, deg0, deg1)

    acc0, acc1 = _edge_kernel(src01, dst2, h2flat, zeros_acc)

    out = pl.pallas_call(
        _combine_body,
        grid=(_NBLK_TC,),
        in_specs=[
            pl.BlockSpec((_RB, H), lambda i: (i, 0)),
            pl.BlockSpec((_RB, H), lambda i: (i, 0)),
            pl.BlockSpec((_RB, H), lambda i: (i, 0)),
            pl.BlockSpec((_RB, H), lambda i: (i + _NBLK_TC, 0)),
            pl.BlockSpec((_RB, 1), lambda i: (i, 0)),
            pl.BlockSpec((1, H), lambda i: (0, 0)),
            pl.BlockSpec((1, H), lambda i: (0, 0)),
        ],
        out_specs=pl.BlockSpec((_RB, D_OUT), lambda i: (i, 0)),
        out_shape=jax.ShapeDtypeStruct((N_NODES, D_OUT), jnp.float32),
    )(acc0, acc1, h2flat, h2flat, dis,
      b[:H].reshape(1, H), b[H:].reshape(1, H))
    return out


# speedup vs baseline: 34.0393x; 1.2912x over previous
import functools

import jax
import jax.numpy as jnp
from jax import lax
from jax.experimental import pallas as pl
from jax.experimental.pallas import tpu as pltpu
from jax.experimental.pallas import tpu_sc as plsc

N_NODES = 50000
N_EDGES = 800000
D_IN = 64
D_OUT = 64
H = 32
NC, NS = 2, 16
LANE = 128
EROWS = 6400
EPAD = EROWS * LANE - N_EDGES
NPAD = 50048
SCRAP = NPAD - 1
SROWS = NPAD // NS
DEG_W = 8
TROWS = EROWS // NS
BLK = 8
NBODY = TROWS // (2 * BLK)
DROWS = EROWS // (NC * NS)
DNBLK = DROWS // BLK
NRING = 6

_mesh = plsc.VectorSubcoreMesh(
    core_axis_name="c", subcore_axis_name="s", num_cores=NC, num_subcores=NS)
_sc_params = pltpu.CompilerParams(use_tc_tiling_on_sc=False)


@functools.partial(
    pl.kernel,
    out_type=(jax.ShapeDtypeStruct((NPAD, DEG_W), jnp.float32),
              jax.ShapeDtypeStruct((NPAD, DEG_W), jnp.float32)),
    mesh=_mesh,
    scratch_types=[
        pltpu.VMEM((BLK, LANE), jnp.int32),
        pltpu.VMEM((LANE, DEG_W), jnp.float32),
        pltpu.SemaphoreType.DMA,
        pltpu.VMEM_SHARED((NPAD, DEG_W), jnp.float32),
    ],
    compiler_params=_sc_params,
)
def _deg_kernel(dst_hbm, ones_hbm, zeros_hbm, deg0_hbm, deg1_hbm,
                dstb, ones_v, sem, deg_s):
    c = lax.axis_index("c")
    s = lax.axis_index("s")
    wid = c * NS + s
    base_rows = s * SROWS
    pltpu.sync_copy(ones_hbm, ones_v)
    pltpu.sync_copy(zeros_hbm, deg_s.at[pl.ds(base_rows, SROWS)])
    plsc.subcore_barrier()
    wbase = wid * DROWS

    def body(k, carry):
        pltpu.sync_copy(dst_hbm.at[pl.ds(wbase + k * BLK, BLK)], dstb)
        descs = [
            pltpu.async_copy(ones_v, deg_s.at[dstb.at[j]], sem, add=True)
            for j in range(BLK)
        ]
        for d in descs:
            d.wait()
        return carry

    lax.fori_loop(0, DNBLK, body, 0)
    plsc.subcore_barrier()

    @pl.when(c == 0)
    def _():
        pltpu.sync_copy(deg_s.at[pl.ds(base_rows, SROWS)],
                        deg0_hbm.at[pl.ds(base_rows, SROWS)])

    @pl.when(c == 1)
    def _():
        pltpu.sync_copy(deg_s.at[pl.ds(base_rows, SROWS)],
                        deg1_hbm.at[pl.ds(base_rows, SROWS)])


@functools.partial(
    pl.kernel,
    out_type=(jax.ShapeDtypeStruct((NPAD, H), jnp.bfloat16),
              jax.ShapeDtypeStruct((NPAD, H), jnp.bfloat16)),
    mesh=_mesh,
    scratch_types=[
        [pltpu.VMEM((BLK, LANE), jnp.int32)] * 2,
        [pltpu.VMEM((BLK, LANE), jnp.int32)] * 2,
        [pltpu.VMEM((LANE, H), jnp.bfloat16)] * NRING,
        [pltpu.SemaphoreType.DMA] * 2,
        [pltpu.SemaphoreType.DMA] * NRING,
        [pltpu.SemaphoreType.DMA] * NRING,
        pltpu.VMEM_SHARED((NPAD, H), jnp.bfloat16),
    ],
    compiler_params=_sc_params,
)
def _edge_kernel(src_hbm, dst_hbm, h2_hbm, zeros_hbm, acc0_hbm, acc1_hbm,
                 srcb, dstb, rows, isems, gsems, ssems, acc_s):
    c = lax.axis_index("c")
    s = lax.axis_index("s")
    base_rows = s * SROWS
    pltpu.sync_copy(zeros_hbm, acc_s.at[pl.ds(base_rows, SROWS)])
    plsc.subcore_barrier()
    tb = s * TROWS

    def iload(blk_idx, buf):
        base = blk_idx * BLK
        a = pltpu.async_copy(
            src_hbm.at[pl.ds(c * EROWS + tb + base, BLK)], srcb[buf],
            isems[buf])
        b = pltpu.async_copy(
            dst_hbm.at[pl.ds(tb + base, BLK)], dstb[buf], isems[buf])
        return a, b

    def iwait(buf):
        pltpu.make_async_copy(
            src_hbm.at[pl.ds(tb, BLK)], srcb[buf], isems[buf]).wait()
        pltpu.make_async_copy(
            dst_hbm.at[pl.ds(tb, BLK)], dstb[buf], isems[buf]).wait()

    iload(0, 0)
    iload(1, 1)

    def body(m, carry):
        iwait(0)
        gd = [None] * NRING
        sd = [None] * NRING
        nrows = 2 * BLK
        for t in range(nrows + NRING - 1):
            slot = t % NRING
            if t == BLK:
                iwait(1)
            if t < nrows:
                if sd[slot] is not None:
                    sd[slot].wait()
                    sd[slot] = None
                srow = srcb[0].at[t] if t < BLK else srcb[1].at[t - BLK]
                gd[slot] = pltpu.async_copy(
                    h2_hbm.at[srow], rows[slot], gsems[slot])
            if t >= NRING - 1:
                r = t - (NRING - 1)
                rs = r % NRING
                gd[rs].wait()
                drow = dstb[0].at[r] if r < BLK else dstb[1].at[r - BLK]
                sd[rs] = pltpu.async_copy(
                    rows[rs], acc_s.at[drow], ssems[rs], add=True)
            if t == BLK + NRING - 1:
                @pl.when(m < NBODY - 1)
                def _():
                    iload(2 * m + 2, 0)
        for slot in range(NRING):
            if sd[slot] is not None:
                sd[slot].wait()

        @pl.when(m < NBODY - 1)
        def _():
            iload(2 * m + 3, 1)
        return carry

    lax.fori_loop(0, NBODY, body, 0)
    plsc.subcore_barrier()

    @pl.when(c == 0)
    def _():
        pltpu.sync_copy(acc_s.at[pl.ds(base_rows, SROWS)],
                        acc0_hbm.at[pl.ds(base_rows, SROWS)])

    @pl.when(c == 1)
    def _():
        pltpu.sync_copy(acc_s.at[pl.ds(base_rows, SROWS)],
                        acc1_hbm.at[pl.ds(base_rows, SROWS)])


_RB = 2000
_NBLK_TC = N_NODES // _RB


def _mm_body(x_ref, w_ref, d0_ref, d1_ref, h2_ref, dis_ref):
    deg = d0_ref[:, 0:1] + d1_ref[:, 0:1] + 1.0
    dis = lax.rsqrt(deg)
    h = jnp.dot(x_ref[...], w_ref[0], preferred_element_type=jnp.float32)
    h2_ref[...] = (h * dis).astype(jnp.bfloat16)
    dis_ref[...] = dis


def _combine_body(a0_ref, a1_ref, h2a_ref, h2b_ref, dis_ref,
                  b0_ref, b1_ref, out_ref):
    dis = dis_ref[...]
    a0 = a0_ref[...].astype(jnp.float32)
    a1 = a1_ref[...].astype(jnp.float32)
    h2a = h2a_ref[...].astype(jnp.float32)
    h2b = h2b_ref[...].astype(jnp.float32)
    o0 = dis * (a0 + h2a) + b0_ref[...]
    o1 = dis * (a1 + h2b) + b1_ref[...]
    out_ref[...] = jnp.concatenate([o0, o1], axis=1)


def kernel(x, edge_index, W, b):
    src_pad = jnp.concatenate(
        [edge_index[0], jnp.zeros((EPAD,), jnp.int32)])
    dst_pad = jnp.concatenate(
        [edge_index[1], jnp.full((EPAD,), SCRAP, jnp.int32)])
    src01 = jnp.concatenate(
        [src_pad, src_pad + N_NODES]).reshape(NC * EROWS, LANE)
    dst2 = dst_pad.reshape(EROWS, LANE)
    wsplit = jnp.stack([W[:, :H], W[:, H:]])
    ones_deg = jnp.ones((LANE, DEG_W), jnp.float32)
    zeros_deg = jnp.zeros((SROWS, DEG_W), jnp.float32)
    zeros_acc = jnp.zeros((SROWS, H), jnp.bfloat16)

    deg0, deg1 = _deg_kernel(dst2, ones_deg, zeros_deg)

    h2flat, dis = pl.pallas_call(
        _mm_body,
        grid=(2, _NBLK_TC),
        in_specs=[
            pl.BlockSpec((_RB, D_IN), lambda p, i: (i, 0)),
            pl.BlockSpec((1, D_IN, H), lambda p, i: (p, 0, 0)),
            pl.BlockSpec((_RB, DEG_W), lambda p, i: (i, 0)),
            pl.BlockSpec((_RB, DEG_W), lambda p, i: (i, 0)),
        ],
        out_specs=[
            pl.BlockSpec((_RB, H), lambda p, i: (p * _NBLK_TC + i, 0)),
            pl.BlockSpec((_RB, 1), lambda p, i: (i, 0)),
        ],
        out_shape=[
            jax.ShapeDtypeStruct((2 * N_NODES, H), jnp.bfloat16),
            jax.ShapeDtypeStruct((N_NODES, 1), jnp.float32),
        ],
    )(x, wsplit, deg0, deg1)

    acc0, acc1 = _edge_kernel(src01, dst2, h2flat, zeros_acc)

    out = pl.pallas_call(
        _combine_body,
        grid=(_NBLK_TC,),
        in_specs=[
            pl.BlockSpec((_RB, H), lambda i: (i, 0)),
            pl.BlockSpec((_RB, H), lambda i: (i, 0)),
            pl.BlockSpec((_RB, H), lambda i: (i, 0)),
            pl.BlockSpec((_RB, H), lambda i: (i + _NBLK_TC, 0)),
            pl.BlockSpec((_RB, 1), lambda i: (i, 0)),
            pl.BlockSpec((1, H), lambda i: (0, 0)),
            pl.BlockSpec((1, H), lambda i: (0, 0)),
        ],
        out_specs=pl.BlockSpec((_RB, D_OUT), lambda i: (i, 0)),
        out_shape=jax.ShapeDtypeStruct((N_NODES, D_OUT), jnp.float32),
    )(acc0, acc1, h2flat, h2flat, dis,
      b[:H].reshape(1, H), b[H:].reshape(1, H))
    return out
